# baseline (speedup 1.0000x reference)
import functools

import jax
import jax.numpy as jnp
from jax import lax
from jax.experimental import pallas as pl
from jax.experimental.pallas import tpu as pltpu
from jax.experimental.pallas import tpu_sc as plsc

N_TOK = 16384
HIDDEN = 2048
E = 16
K = 2
BT = 1024
GRID = N_TOK // BT

_SC_INFO = plsc.get_sparse_core_info()
NC = _SC_INFO.num_cores
NS = _SC_INFO.num_subcores
L = _SC_INFO.num_lanes
NW = NC * NS
CH = N_TOK // NW
SLABS_PER_STEP = BT // CH


def _gate_kernel(x_ref, w_ref, lg_ref, z_ref):
    i = pl.program_id(0)
    w = w_ref[...]
    logits = lax.dot_general(
        w, x_ref[...], (((1,), (1,)), ((), ())),
        preferred_element_type=jnp.float32,
    )

    part = jnp.sum(logits * logits)

    @pl.when(i == 0)
    def _():
        z_ref[0] = 0.0

    z_ref[0] += part

    for s in range(SLABS_PER_STEP):
        lg_ref[s] = logits[:, s * CH:(s + 1) * CH]


def _route_sc(lg_hbm, i1_hbm, i2_hbm, s1_hbm, s2_hbm,
              buf, oi1, oi2, os1, os2):
    wid = lax.axis_index("s") * NC + lax.axis_index("c")
    pltpu.sync_copy(lg_hbm.at[wid], buf)

    def body(g, carry):
        base = g * L
        neg = jnp.full((L,), -jnp.inf, jnp.float32)
        m1 = neg
        m2 = neg
        i1 = jnp.zeros((L,), jnp.int32)
        i2 = jnp.zeros((L,), jnp.int32)
        for e in range(E):
            v = buf[e, pl.ds(base, L)]
            gt1 = v > m1
            gt2 = v > m2
            m2 = jnp.where(gt1, m1, jnp.where(gt2, v, m2))
            i2 = jnp.where(gt1, i1, jnp.where(gt2, e, i2))
            m1 = jnp.where(gt1, v, m1)
            i1 = jnp.where(gt1, e, i1)
        den = jnp.zeros((L,), jnp.float32)
        for e in range(E):
            v = buf[e, pl.ds(base, L)]
            den = den + jnp.exp(v - m1)
        oi1[pl.ds(base, L)] = i1
        oi2[pl.ds(base, L)] = i2
        os1[pl.ds(base, L)] = 1.0 / den
        os2[pl.ds(base, L)] = jnp.exp(m2 - m1) / den
        return carry

    lax.fori_loop(0, CH // L, body, 0)

    pltpu.sync_copy(oi1, i1_hbm.at[wid])
    pltpu.sync_copy(oi2, i2_hbm.at[wid])
    pltpu.sync_copy(os1, s1_hbm.at[wid])
    pltpu.sync_copy(os2, s2_hbm.at[wid])


_route_call = functools.partial(
    pl.kernel,
    mesh=plsc.VectorSubcoreMesh(core_axis_name="c", subcore_axis_name="s"),
    out_type=[
        jax.ShapeDtypeStruct((NW, CH), jnp.int32),
        jax.ShapeDtypeStruct((NW, CH), jnp.int32),
        jax.ShapeDtypeStruct((NW, CH), jnp.float32),
        jax.ShapeDtypeStruct((NW, CH), jnp.float32),
    ],
    scratch_types=[
        pltpu.VMEM((E, CH), jnp.float32),
        pltpu.VMEM((CH,), jnp.int32),
        pltpu.VMEM((CH,), jnp.int32),
        pltpu.VMEM((CH,), jnp.float32),
        pltpu.VMEM((CH,), jnp.float32),
    ],
)(_route_sc)


def kernel(x, W):
    lg, zsum = pl.pallas_call(
        _gate_kernel,
        grid=(GRID,),
        in_specs=[
            pl.BlockSpec((BT, HIDDEN), lambda i: (i, 0)),
            pl.BlockSpec((E, HIDDEN), lambda i: (0, 0)),
        ],
        out_specs=[
            pl.BlockSpec((SLABS_PER_STEP, E, CH), lambda i: (i, 0, 0)),
            pl.BlockSpec(memory_space=pltpu.SMEM),
        ],
        out_shape=[
            jax.ShapeDtypeStruct((NW, E, CH), jnp.float32),
            jax.ShapeDtypeStruct((1,), jnp.float32),
        ],
    )(x, W)

    i1, i2, s1, s2 = _route_call(lg)

    idx = jnp.stack([i1.reshape(N_TOK), i2.reshape(N_TOK)], axis=-1)
    scores = jnp.stack([s1.reshape(N_TOK), s2.reshape(N_TOK)], axis=-1)
    z_loss = zsum[0] / jnp.float32(N_TOK * E)
    aux_loss = jnp.zeros((), jnp.float32)
    return (idx, scores, aux_loss, z_loss)
